# baseline (speedup 1.0000x reference)
import functools
import math

import jax
import jax.numpy as jnp
import numpy as np
from jax import lax
from jax.experimental import pallas as pl
from jax.experimental.pallas import tpu as pltpu
from jax.experimental.pallas import tpu_sc as plsc

NUM_USER = 100000
NUM_ITEM = 1000000
DIM_E = 64
NUM_NEG = 4
TEMP_VALUE = 0.2
LR_LAMBDA = 0.5
NUM_SAMPLE = 0.5
BATCH = 4096

B = BATCH * (1 + NUM_NEG)
NW = 32
BPW = B // NW
CHUNK = 128
NCHUNK = BPW // CHUNK
NBLK = BPW // 16
LOG5 = float(math.log(5.0))

_mesh = plsc.VectorSubcoreMesh(core_axis_name="c", subcore_axis_name="s")


@functools.partial(
    pl.kernel,
    out_type=(
        jax.ShapeDtypeStruct((B,), jnp.float32),
        jax.ShapeDtypeStruct((B,), jnp.float32),
        jax.ShapeDtypeStruct((B,), jnp.float32),
    ),
    mesh=_mesh,
    compiler_params=pltpu.CompilerParams(use_tc_tiling_on_sc=False),
    scratch_types=[
        pltpu.VMEM((BPW,), jnp.int32),
        pltpu.VMEM((BPW,), jnp.int32),
        pltpu.VMEM((BPW,), jnp.float32),
        pltpu.VMEM((BPW, DIM_E), jnp.float32),
        pltpu.VMEM((BPW, DIM_E), jnp.float32),
        pltpu.VMEM((BPW,), jnp.float32),
        pltpu.VMEM((BPW,), jnp.float32),
        pltpu.VMEM((BPW,), jnp.float32),
        pltpu.SemaphoreType.DMA,
        pltpu.SemaphoreType.DMA,
    ],
)
def _sc_dot_kernel(emb_hbm, uidx_hbm, vidx_hbm, keep_hbm,
                   dot_hbm, usq_hbm, vsq_hbm,
                   uidx_v, vidx_v, keep_v, urows_v, vrows_v,
                   dot_v, usq_v, vsq_v, usem, vsem):
    wid = lax.axis_index("s") * 2 + lax.axis_index("c")
    base = wid * BPW

    pltpu.sync_copy(uidx_hbm.at[pl.ds(base, BPW)], uidx_v)
    pltpu.sync_copy(vidx_hbm.at[pl.ds(base, BPW)], vidx_v)
    pltpu.sync_copy(keep_hbm.at[pl.ds(base, BPW)], keep_v)

    copies = []
    for c in range(NCHUNK):
        sl = pl.ds(c * CHUNK, CHUNK)
        copies.append(pltpu.async_copy(
            emb_hbm.at[uidx_v.at[sl]], urows_v.at[sl], usem))
        copies.append(pltpu.async_copy(
            emb_hbm.at[vidx_v.at[sl]], vrows_v.at[sl], vsem))
    for h in copies:
        h.wait()

    def blk_body(blk, carry):
        b0 = blk * 16
        lane = lax.iota(jnp.int32, 16)

        dn = lax.GatherDimensionNumbers(
            offset_dims=(), collapsed_slice_dims=(0,), start_index_map=(0,))

        def hsum(x):
            for s in (1, 2, 4, 8):
                x = x + lax.gather(
                    x, (lane ^ s)[:, None], dn, slice_sizes=(1,),
                    mode=lax.GatherScatterMode.PROMISE_IN_BOUNDS)
            return x

        dres = jnp.zeros((16,), jnp.float32)
        ures = jnp.zeros((16,), jnp.float32)
        vres = jnp.zeros((16,), jnp.float32)
        for rr in range(16):
            row = b0 + rr
            sel = lane == rr
            da = jnp.zeros((16,), jnp.float32)
            ua = jnp.zeros((16,), jnp.float32)
            va = jnp.zeros((16,), jnp.float32)
            for kc in range(DIM_E // 16):
                uch = urows_v[row, pl.ds(kc * 16, 16)]
                vch = vrows_v[row, pl.ds(kc * 16, 16)]
                da = da + uch * vch
                ua = ua + uch * uch
                va = va + vch * vch
            dres = jnp.where(sel, hsum(da), dres)
            ures = jnp.where(sel, hsum(ua), ures)
            vres = jnp.where(sel, hsum(va), vres)
        keep16 = keep_v[pl.ds(b0, 16)]
        dot_v[pl.ds(b0, 16)] = dres * keep16
        usq_v[pl.ds(b0, 16)] = ures
        vsq_v[pl.ds(b0, 16)] = vres
        return carry

    lax.fori_loop(0, NBLK, blk_body, 0)

    pltpu.sync_copy(dot_v, dot_hbm.at[pl.ds(base, BPW)])
    pltpu.sync_copy(usq_v, usq_hbm.at[pl.ds(base, BPW)])
    pltpu.sync_copy(vsq_v, vsq_hbm.at[pl.ds(base, BPW)])


def _finalize_body(dott_ref, usq_ref, vsq_ref, total_ref, reg_ref):
    d = dott_ref[...]
    s = jnp.exp(d * (1.0 / TEMP_VALUE))
    pos = s[0:1, :]
    tot = jnp.sum(s, axis=0, keepdims=True)
    loss2 = jnp.sum(-jnp.log(pos / tot)) / float(BATCH)
    ru = jnp.sum(jnp.sqrt(usq_ref[...])) / float(B)
    rv = jnp.sum(jnp.sqrt(vsq_ref[...])) / float(B)
    total_ref[...] = jnp.reshape(
        LOG5 * LR_LAMBDA + loss2 * (1.0 - LR_LAMBDA), (1, 1))
    reg_ref[...] = jnp.reshape((ru + rv) * 0.5, (1, 1))


def kernel(user_tensor, item_tensor, id_embedding):
    user_flat = user_tensor.reshape(-1).astype(jnp.int32)
    item_flat = item_tensor.reshape(-1).astype(jnp.int32)

    n = B
    rand_index = jax.random.randint(
        jax.random.key(42), (int(n * NUM_SAMPLE),), 0, n)
    keep = jnp.ones((B,), jnp.float32).at[rand_index].set(0.0)

    dot, usq, vsq = _sc_dot_kernel(id_embedding, user_flat, item_flat, keep)

    dott = dot.reshape(BATCH, 1 + NUM_NEG).T
    usq2 = usq.reshape(B // 128, 128)
    vsq2 = vsq.reshape(B // 128, 128)

    total, reg = pl.pallas_call(
        _finalize_body,
        out_shape=(
            jax.ShapeDtypeStruct((1, 1), jnp.float32),
            jax.ShapeDtypeStruct((1, 1), jnp.float32),
        ),
    )(dott, usq2, vsq2)
    return (total[0, 0], reg[0, 0])


# speedup vs baseline: 1.4396x; 1.4396x over previous
import functools
import math

import jax
import jax.numpy as jnp
import numpy as np
from jax import lax
from jax.experimental import pallas as pl
from jax.experimental.pallas import tpu as pltpu
from jax.experimental.pallas import tpu_sc as plsc

NUM_USER = 100000
NUM_ITEM = 1000000
DIM_E = 64
NUM_NEG = 4
TEMP_VALUE = 0.2
LR_LAMBDA = 0.5
NUM_SAMPLE = 0.5
BATCH = 4096

B = BATCH * (1 + NUM_NEG)
NW = 32
BPW = B // NW
CHUNK = 128
NCHUNK = BPW // CHUNK
NBLK = BPW // 16
LOG5 = float(math.log(5.0))

_mesh = plsc.VectorSubcoreMesh(core_axis_name="c", subcore_axis_name="s")


@functools.partial(
    pl.kernel,
    out_type=(
        jax.ShapeDtypeStruct((B,), jnp.float32),
        jax.ShapeDtypeStruct((B,), jnp.float32),
        jax.ShapeDtypeStruct((B,), jnp.float32),
    ),
    mesh=_mesh,
    scratch_types=[
        pltpu.VMEM((BPW,), jnp.int32),
        pltpu.VMEM((BPW,), jnp.int32),
        pltpu.VMEM((BPW,), jnp.float32),
        pltpu.VMEM((BPW, 2 * DIM_E), jnp.float32),
        pltpu.VMEM((BPW,), jnp.float32),
        pltpu.VMEM((BPW,), jnp.float32),
        pltpu.VMEM((BPW,), jnp.float32),
        pltpu.SemaphoreType.DMA,
        pltpu.SemaphoreType.DMA,
    ],
)
def _sc_dot_kernel(emb_hbm, uidx_hbm, vidx_hbm, keep_hbm,
                   dot_hbm, usq_hbm, vsq_hbm,
                   uidx_v, vidx_v, keep_v, rows_v,
                   dot_v, usq_v, vsq_v, usem, vsem):
    wid = lax.axis_index("s") * 2 + lax.axis_index("c")
    base = wid * BPW

    pltpu.sync_copy(uidx_hbm.at[pl.ds(base, BPW)], uidx_v)
    pltpu.sync_copy(vidx_hbm.at[pl.ds(base, BPW)], vidx_v)
    pltpu.sync_copy(keep_hbm.at[pl.ds(base, BPW)], keep_v)

    def issue_group(g, carry):
        g0 = g * 16
        uvec = uidx_v[pl.ds(g0, 16)]
        vvec = vidx_v[pl.ds(g0, 16)]
        for rr in range(16):
            pltpu.make_async_copy(
                emb_hbm.at[uvec[rr]],
                rows_v.at[g0 + rr, pl.ds(0, DIM_E)], usem).start()
            pltpu.make_async_copy(
                emb_hbm.at[vvec[rr]],
                rows_v.at[g0 + rr, pl.ds(DIM_E, DIM_E)], vsem).start()
        return carry

    lax.fori_loop(0, NBLK, issue_group, 0)

    def drain_row(i, carry):
        pltpu.make_async_copy(
            emb_hbm.at[0], rows_v.at[i, pl.ds(0, DIM_E)], usem).wait()
        pltpu.make_async_copy(
            emb_hbm.at[0], rows_v.at[i, pl.ds(DIM_E, DIM_E)], vsem).wait()
        return carry

    lax.fori_loop(0, BPW, drain_row, 0)

    def blk_body(blk, carry):
        b0 = blk * 16
        lane = lax.iota(jnp.int32, 16)

        dn = lax.GatherDimensionNumbers(
            offset_dims=(), collapsed_slice_dims=(0,), start_index_map=(0,))

        def hsum(x):
            for s in (1, 2, 4, 8):
                x = x + lax.gather(
                    x, (lane ^ s)[:, None], dn, slice_sizes=(1,),
                    mode=lax.GatherScatterMode.PROMISE_IN_BOUNDS)
            return x

        dres = jnp.zeros((16,), jnp.float32)
        ures = jnp.zeros((16,), jnp.float32)
        vres = jnp.zeros((16,), jnp.float32)
        for rr in range(16):
            row = b0 + rr
            sel = lane == rr
            da = jnp.zeros((16,), jnp.float32)
            ua = jnp.zeros((16,), jnp.float32)
            va = jnp.zeros((16,), jnp.float32)
            for kc in range(DIM_E // 16):
                uch = rows_v[row, pl.ds(kc * 16, 16)]
                vch = rows_v[row, pl.ds(DIM_E + kc * 16, 16)]
                da = da + uch * vch
                ua = ua + uch * uch
                va = va + vch * vch
            dres = jnp.where(sel, hsum(da), dres)
            ures = jnp.where(sel, hsum(ua), ures)
            vres = jnp.where(sel, hsum(va), vres)
        keep16 = keep_v[pl.ds(b0, 16)]
        dot_v[pl.ds(b0, 16)] = dres * keep16
        usq_v[pl.ds(b0, 16)] = ures
        vsq_v[pl.ds(b0, 16)] = vres
        return carry

    lax.fori_loop(0, NBLK, blk_body, 0)

    pltpu.sync_copy(dot_v, dot_hbm.at[pl.ds(base, BPW)])
    pltpu.sync_copy(usq_v, usq_hbm.at[pl.ds(base, BPW)])
    pltpu.sync_copy(vsq_v, vsq_hbm.at[pl.ds(base, BPW)])


def _finalize_body(dott_ref, usq_ref, vsq_ref, total_ref, reg_ref):
    d = dott_ref[...]
    s = jnp.exp(d * (1.0 / TEMP_VALUE))
    pos = s[0:1, :]
    tot = jnp.sum(s, axis=0, keepdims=True)
    loss2 = jnp.sum(-jnp.log(pos / tot)) / float(BATCH)
    ru = jnp.sum(jnp.sqrt(usq_ref[...])) / float(B)
    rv = jnp.sum(jnp.sqrt(vsq_ref[...])) / float(B)
    total_ref[...] = jnp.reshape(
        LOG5 * LR_LAMBDA + loss2 * (1.0 - LR_LAMBDA), (1, 1))
    reg_ref[...] = jnp.reshape((ru + rv) * 0.5, (1, 1))


def kernel(user_tensor, item_tensor, id_embedding):
    user_flat = user_tensor.reshape(-1).astype(jnp.int32)
    item_flat = item_tensor.reshape(-1).astype(jnp.int32)

    n = B
    rand_index = jax.random.randint(
        jax.random.key(42), (int(n * NUM_SAMPLE),), 0, n)
    keep = jnp.ones((B,), jnp.float32).at[rand_index].set(0.0)

    dot, usq, vsq = _sc_dot_kernel(id_embedding, user_flat, item_flat, keep)

    dott = dot.reshape(BATCH, 1 + NUM_NEG).T
    usq2 = usq.reshape(B // 128, 128)
    vsq2 = vsq.reshape(B // 128, 128)

    total, reg = pl.pallas_call(
        _finalize_body,
        out_shape=(
            jax.ShapeDtypeStruct((1, 1), jnp.float32),
            jax.ShapeDtypeStruct((1, 1), jnp.float32),
        ),
    )(dott, usq2, vsq2)
    return (total[0, 0], reg[0, 0])


# speedup vs baseline: 1.5855x; 1.1014x over previous
import functools
import math

import jax
import jax.numpy as jnp
import numpy as np
from jax import lax
from jax.experimental import pallas as pl
from jax.experimental.pallas import tpu as pltpu
from jax.experimental.pallas import tpu_sc as plsc

NUM_USER = 100000
NUM_ITEM = 1000000
DIM_E = 64
NUM_NEG = 4
TEMP_VALUE = 0.2
LR_LAMBDA = 0.5
NUM_SAMPLE = 0.5
BATCH = 4096

B = BATCH * (1 + NUM_NEG)
NW = 32
BPW = B // NW
CHUNK = 128
NCHUNK = BPW // CHUNK
NBLK = BPW // 16
LOG5 = float(math.log(5.0))

_mesh = plsc.VectorSubcoreMesh(core_axis_name="c", subcore_axis_name="s")

_RAND_INDEX = np.asarray(
    jax.random.randint(jax.random.key(42), (int(B * NUM_SAMPLE),), 0, B))
_KEEP_HOST = np.ones((B,), np.float32)
_KEEP_HOST[_RAND_INDEX] = 0.0


@functools.partial(
    pl.kernel,
    out_type=(
        jax.ShapeDtypeStruct((B,), jnp.float32),
        jax.ShapeDtypeStruct((B,), jnp.float32),
        jax.ShapeDtypeStruct((B,), jnp.float32),
    ),
    mesh=_mesh,
    scratch_types=[
        pltpu.VMEM((BPW,), jnp.int32),
        pltpu.VMEM((BPW,), jnp.int32),
        pltpu.VMEM((BPW,), jnp.float32),
        pltpu.VMEM((BPW, 2 * DIM_E), jnp.float32),
        pltpu.VMEM((BPW,), jnp.float32),
        pltpu.VMEM((BPW,), jnp.float32),
        pltpu.VMEM((BPW,), jnp.float32),
        pltpu.SemaphoreType.DMA,
        pltpu.SemaphoreType.DMA,
    ],
)
def _sc_dot_kernel(emb_hbm, uidx_hbm, vidx_hbm, keep_hbm,
                   dot_hbm, usq_hbm, vsq_hbm,
                   uidx_v, vidx_v, keep_v, rows_v,
                   dot_v, usq_v, vsq_v, usem, vsem):
    wid = lax.axis_index("s") * 2 + lax.axis_index("c")
    base = wid * BPW

    pltpu.sync_copy(uidx_hbm.at[pl.ds(base, BPW)], uidx_v)
    pltpu.sync_copy(vidx_hbm.at[pl.ds(base, BPW)], vidx_v)
    pltpu.sync_copy(keep_hbm.at[pl.ds(base, BPW)], keep_v)

    def issue_group(g, carry):
        g0 = g * 16
        uvec = uidx_v[pl.ds(g0, 16)]
        vvec = vidx_v[pl.ds(g0, 16)]
        for rr in range(16):
            pltpu.make_async_copy(
                emb_hbm.at[uvec[rr]],
                rows_v.at[g0 + rr, pl.ds(0, DIM_E)], usem).start()
            pltpu.make_async_copy(
                emb_hbm.at[vvec[rr]],
                rows_v.at[g0 + rr, pl.ds(DIM_E, DIM_E)], vsem).start()
        return carry

    lax.fori_loop(0, NBLK, issue_group, 0)

    def drain_row(i, carry):
        pltpu.make_async_copy(
            emb_hbm.at[0], rows_v.at[i, pl.ds(0, DIM_E)], usem).wait()
        pltpu.make_async_copy(
            emb_hbm.at[0], rows_v.at[i, pl.ds(DIM_E, DIM_E)], vsem).wait()
        return carry

    lax.fori_loop(0, BPW, drain_row, 0)

    def blk_body(blk, carry):
        b0 = blk * 16
        lane = lax.iota(jnp.int32, 16)

        dn = lax.GatherDimensionNumbers(
            offset_dims=(), collapsed_slice_dims=(0,), start_index_map=(0,))

        def hsum(x):
            for s in (1, 2, 4, 8):
                x = x + lax.gather(
                    x, (lane ^ s)[:, None], dn, slice_sizes=(1,),
                    mode=lax.GatherScatterMode.PROMISE_IN_BOUNDS)
            return x

        dres = jnp.zeros((16,), jnp.float32)
        ures = jnp.zeros((16,), jnp.float32)
        vres = jnp.zeros((16,), jnp.float32)
        for rr in range(16):
            row = b0 + rr
            sel = lane == rr
            da = jnp.zeros((16,), jnp.float32)
            ua = jnp.zeros((16,), jnp.float32)
            va = jnp.zeros((16,), jnp.float32)
            for kc in range(DIM_E // 16):
                uch = rows_v[row, pl.ds(kc * 16, 16)]
                vch = rows_v[row, pl.ds(DIM_E + kc * 16, 16)]
                da = da + uch * vch
                ua = ua + uch * uch
                va = va + vch * vch
            dres = jnp.where(sel, hsum(da), dres)
            ures = jnp.where(sel, hsum(ua), ures)
            vres = jnp.where(sel, hsum(va), vres)
        keep16 = keep_v[pl.ds(b0, 16)]
        dot_v[pl.ds(b0, 16)] = dres * keep16
        usq_v[pl.ds(b0, 16)] = ures
        vsq_v[pl.ds(b0, 16)] = vres
        return carry

    lax.fori_loop(0, NBLK, blk_body, 0)

    pltpu.sync_copy(dot_v, dot_hbm.at[pl.ds(base, BPW)])
    pltpu.sync_copy(usq_v, usq_hbm.at[pl.ds(base, BPW)])
    pltpu.sync_copy(vsq_v, vsq_hbm.at[pl.ds(base, BPW)])


def _finalize_body(dott_ref, usq_ref, vsq_ref, total_ref, reg_ref):
    d = dott_ref[...]
    s = jnp.exp(d * (1.0 / TEMP_VALUE))
    pos = s[0:1, :]
    tot = jnp.sum(s, axis=0, keepdims=True)
    loss2 = jnp.sum(-jnp.log(pos / tot)) / float(BATCH)
    ru = jnp.sum(jnp.sqrt(usq_ref[...])) / float(B)
    rv = jnp.sum(jnp.sqrt(vsq_ref[...])) / float(B)
    total_ref[...] = jnp.reshape(
        LOG5 * LR_LAMBDA + loss2 * (1.0 - LR_LAMBDA), (1, 1))
    reg_ref[...] = jnp.reshape((ru + rv) * 0.5, (1, 1))


def kernel(user_tensor, item_tensor, id_embedding):
    user_flat = user_tensor.reshape(-1).astype(jnp.int32)
    item_flat = item_tensor.reshape(-1).astype(jnp.int32)

    keep = jnp.asarray(_KEEP_HOST)

    dot, usq, vsq = _sc_dot_kernel(id_embedding, user_flat, item_flat, keep)

    dott = dot.reshape(BATCH, 1 + NUM_NEG).T
    usq2 = usq.reshape(B // 128, 128)
    vsq2 = vsq.reshape(B // 128, 128)

    total, reg = pl.pallas_call(
        _finalize_body,
        out_shape=(
            jax.ShapeDtypeStruct((1, 1), jnp.float32),
            jax.ShapeDtypeStruct((1, 1), jnp.float32),
        ),
    )(dott, usq2, vsq2)
    return (total[0, 0], reg[0, 0])
